# baseline (speedup 1.0000x reference)
import functools

import jax
import jax.numpy as jnp
from jax import lax
from jax.experimental import pallas as pl
from jax.experimental.pallas import tpu as pltpu

KC = 8192
DC = 256

_WINDOWS = ((0, 4096), (4096, 8192))
_SUB = 1024


def _make_argmin_body(assoc):
    def _argmin_body(z_ref, zsum_ref, whi_ref, wlo_ref, wsum_ref, idx_ref):
        z2 = z_ref[0]
        tm = z2.shape[1]
        zsum = zsum_ref[0]
        dn = (((1,), (0,)), ((), ()))
        acc_v = acc_i = None
        for t, (lo, hi) in enumerate(_WINDOWS):
            wmin = wargs = None
            for s in range(lo, hi, _SUB):
                n = min(_SUB, hi - s)
                wsum = wsum_ref[s:s + n, :]
                h = lax.dot_general(whi_ref[s:s + n, :], z2, dn,
                                    preferred_element_type=jnp.float32)
                l = lax.dot_general(wlo_ref[s:s + n, :], z2, dn,
                                    preferred_element_type=jnp.float32)
                if assoc == 0:
                    d = (zsum + wsum) - (h + l)
                else:
                    d = ((zsum + wsum) - h) - l
                smin = jnp.min(d, axis=0, keepdims=True)
                iota = lax.broadcasted_iota(jnp.int32, (n, tm), 0) + s
                sarg = jnp.min(jnp.where(d == smin, iota, KC), axis=0,
                               keepdims=True)
                if s == lo:
                    wmin, wargs = smin, sarg
                else:
                    better = smin < wmin
                    wargs = jnp.where(better, sarg, wargs)
                    wmin = jnp.where(better, smin, wmin)
            if t == 0:
                acc_v = wmin.astype(jnp.bfloat16).astype(jnp.float32)
                acc_i = wargs
            else:
                take = wmin < acc_v
                acc_i = jnp.where(take, wargs, acc_i)
        idx_ref[0] = acc_i
    return _argmin_body


def _encode_indices(z2b, zsum, w_hi, w_lo, wsum, assoc=0):
    b, _, p = z2b.shape
    return pl.pallas_call(
        _make_argmin_body(assoc),
        grid=(b,),
        in_specs=[
            pl.BlockSpec((1, DC, p), lambda i: (i, 0, 0)),
            pl.BlockSpec((1, 1, p), lambda i: (i, 0, 0)),
            pl.BlockSpec((KC, DC), lambda i: (0, 0)),
            pl.BlockSpec((KC, DC), lambda i: (0, 0)),
            pl.BlockSpec((KC, 1), lambda i: (0, 0)),
        ],
        out_specs=pl.BlockSpec((1, 1, p), lambda i: (i, 0, 0)),
        out_shape=jax.ShapeDtypeStruct((b, 1, p), jnp.int32),
    )(z2b, zsum, w_hi, w_lo, wsum)


def _prep(z_e, W):
    B, Dd, H, Wd = z_e.shape
    P = H * Wd
    z2b = (2.0 * z_e).astype(jnp.bfloat16).reshape(B, Dd, P)
    zsum = jnp.sum(z_e * z_e, axis=1).reshape(B, 1, P)
    w_hi = W.astype(jnp.bfloat16)
    w_lo = (W - w_hi.astype(jnp.float32)).astype(jnp.bfloat16)
    wsum = jnp.sum(W * W, axis=1).reshape(KC, 1)
    return z2b, zsum, w_hi, w_lo, wsum


def kernel(z_e, W):
    z_e = z_e.astype(jnp.float32)
    B, Dd, H, Wd = z_e.shape
    P = H * Wd
    z2b, zsum, w_hi, w_lo, wsum = _prep(z_e, W)
    idx3 = _encode_indices(z2b, zsum, w_hi, w_lo, wsum)
    encoding_indices = idx3.reshape(B, H, Wd)
    flat_idx = idx3.reshape(-1)

    z_q_flat = jnp.take(W, flat_idx, axis=0)
    z_q = jnp.transpose(z_q_flat.reshape(B, H, Wd, Dd), (0, 3, 1, 2))
    vq_loss = jnp.mean((z_q - z_e) ** 2)
    commitment_loss = 0.25 * vq_loss
    z_q_st = z_e + (z_q - z_e)
    counts = jnp.bincount(flat_idx, length=KC)
    unique_codes = jnp.sum(counts > 0)
    codebook_usage = unique_codes.astype(jnp.float32) / KC
    avg_probs = counts.astype(jnp.float32) / flat_idx.shape[0]
    perplexity = jnp.exp(-jnp.sum(avg_probs * jnp.log(avg_probs + 1e-10)))
    return (z_q_st, vq_loss, commitment_loss,
            encoding_indices, codebook_usage, perplexity)
